# baseline (speedup 1.0000x reference)
import jax
import jax.numpy as jnp
from jax.experimental import pallas as pl
from jax.experimental.pallas import tpu as pltpu

F32 = jnp.float32
BF16 = jnp.bfloat16
B = 8192


def _s2d(x):
    b, h, w, c = x.shape
    return x.reshape(b, h // 2, 2, w // 2, 2, c).transpose(0, 1, 3, 2, 4, 5).reshape(
        b, h // 2, w // 2, 4 * c)


def _build_m1(W1):
    mats = []
    for Ay in (0, 1):
        for Ax in (0, 1):
            M = jnp.zeros((2, 2, 2, 2, 2, 2, 32), F32)
            for qy in (0, 1):
                for qx in (0, 1):
                    for p0y in (0, 1):
                        for p0x in (0, 1):
                            for py in (0, 1):
                                for px in (0, 1):
                                    ay = 2 * Ay + qy - py
                                    ax = 2 * Ax + qx - px
                                    if ay in (0, 1) and ax in (0, 1):
                                        ky, kx = 2 * ay + p0y, 2 * ax + p0x
                                        if ky <= 2 and kx <= 2:
                                            M = M.at[qy, qx, p0y, p0x, py, px, :].set(
                                                W1[ky, kx, 0, :])
            mats.append(M.reshape(16, 128))
    return jnp.concatenate(mats, axis=0)


def _build_k2(W2):
    mats = []
    for Ay in (0, 1):
        for Ax in (0, 1):
            M = jnp.zeros((2, 2, 32, 64), F32)
            for py in (0, 1):
                for px in (0, 1):
                    ky, kx = 2 * Ay + py, 2 * Ax + px
                    if ky <= 2 and kx <= 2:
                        M = M.at[py, px].set(W2[ky, kx])
            mats.append(M.reshape(128, 64))
    return jnp.concatenate(mats, axis=0)


def _build_d1(DW1):
    mats = []
    for Ay in (0, 1):
        for Ax in (0, 1):
            M = jnp.zeros((32, 2, 2, 64), F32)
            for ry in (0, 1):
                for rx in (0, 1):
                    ky, kx = 2 * Ay - ry, 2 * Ax - rx
                    if 0 <= ky <= 2 and 0 <= kx <= 2:
                        M = M.at[:, ry, rx, :].set(DW1[ky, kx])
            mats.append(M.reshape(32, 256))
    return jnp.concatenate(mats, axis=0)


def _build_m2(DW2):
    mats = []
    for Ay in (0, 1):
        for Ax in (0, 1):
            M = jnp.zeros((64, 2, 2, 32), F32)
            for sy in (0, 1):
                for sx in (0, 1):
                    ky, kx = 2 * Ay - sy, 2 * Ax - sx
                    if 0 <= ky <= 2 and 0 <= kx <= 2:
                        M = M.at[:, sy, sx, :].set(DW2[ky, kx])
            mats.append(M.reshape(64, 128))
    return jnp.concatenate(mats, axis=0)


def _build_m3(W3):
    mats = []
    for ay in (-1, 0, 1):
        for ax in (-1, 0, 1):
            M = jnp.zeros((2, 2, 32, 2, 2), F32)
            for ry in (0, 1):
                for rx in (0, 1):
                    for sy in (0, 1):
                        for sx in (0, 1):
                            dy = 2 * ay + ry + 1 - sy
                            dx = 2 * ax + rx + 1 - sx
                            if 0 <= dy <= 2 and 0 <= dx <= 2:
                                M = M.at[ry, rx, :, sy, sx].set(W3[dy, dx, :, 0])
            mats.append(M.reshape(128, 4))
    return mats


def _enc_conv_kernel(xq_ref, m1_ref, k2_ref, b1_ref, b2_ref, h_ref, s1_ref):
    bb = xq_ref.shape[0]
    sl = [xq_ref[:, ay:ay + 7, ax:ax + 7, :].astype(BF16)
          for ay in (0, 1) for ax in (0, 1)]
    X = jnp.concatenate(sl, axis=-1).reshape(bb * 49, 64)
    o1 = jnp.dot(X, m1_ref[...], preferred_element_type=F32)
    o1 = jnp.maximum(o1 + b1_ref[0], 0.0)
    s1_ref[:, 7:8, :, :] = jnp.zeros((bb, 1, 8, 128), F32)
    s1_ref[:, :, 7:8, :] = jnp.zeros((bb, 8, 1, 128), F32)
    s1_ref[:, :7, :7, :] = o1.reshape(bb, 7, 7, 128)
    sl2 = [s1_ref[:, ay:ay + 7, ax:ax + 7, :].astype(BF16)
           for ay in (0, 1) for ax in (0, 1)]
    X2 = jnp.concatenate(sl2, axis=-1).reshape(bb * 49, 512)
    o2 = jnp.dot(X2, k2_ref[...], preferred_element_type=F32)
    h_ref[...] = jnp.maximum(o2 + b2_ref[0], 0.0).reshape(bb, 7, 7, 64)


def _vq_kernel(flat_ref, encW_ref, encb_ref, embT_ref, emb2_ref, embQ_ref,
               decW_ref, decb_ref, d_ref, loss_ref):
    bb = flat_ref.shape[0]
    enc = jnp.dot(flat_ref[...].astype(BF16), encW_ref[...],
                  preferred_element_type=F32) + encb_ref[0]
    cross = jnp.dot(enc.astype(BF16), embT_ref[...], preferred_element_type=F32)
    dist = jnp.sum(enc * enc, axis=1, keepdims=True) + emb2_ref[0] - 2.0 * cross
    idx = jnp.argmin(dist, axis=1)
    onehot = (jax.lax.broadcasted_iota(jnp.int32, (bb, 64), 1)
              == idx[:, None]).astype(BF16)
    q = jnp.dot(onehot, embQ_ref[...], preferred_element_type=F32)
    diff = q - enc
    part = jnp.sum(diff * diff) * (2.0 / (B * 128))

    @pl.when(pl.program_id(0) == 0)
    def _():
        loss_ref[0, 0] = 0.0

    loss_ref[0, 0] += part
    dd = jnp.dot(q.astype(BF16), decW_ref[...], preferred_element_type=F32)
    d_ref[...] = jnp.maximum(dd + decb_ref[0], 0.0)


def _deconv1_kernel(dp_ref, d1_ref, db1_ref, r1_ref):
    bb = dp_ref.shape[0]
    sl = [dp_ref[:, ay:ay + 7, ax:ax + 7, :].astype(BF16)
          for ay in (0, 1) for ax in (0, 1)]
    X = jnp.concatenate(sl, axis=-1).reshape(bb * 49, 128)
    o = jnp.dot(X, d1_ref[...], preferred_element_type=F32)
    r1_ref[...] = jnp.maximum(o + db1_ref[0], 0.0).reshape(bb, 7, 7, 256)


def _dec_kernel(inp_ref, m2_ref, db2_ref, m3s_ref, b3_ref, out_ref, s_ref):
    bb = inp_ref.shape[0]
    sl = [inp_ref[:, ay:ay + 14, ax:ax + 14, :].astype(BF16)
          for ay in (0, 1) for ax in (0, 1)]
    X = jnp.concatenate(sl, axis=-1).reshape(bb * 196, 256)
    r2 = jnp.dot(X, m2_ref[...], preferred_element_type=F32)
    r2 = jnp.maximum(r2 + db2_ref[0], 0.0)
    s_ref[:, 0:1, :, :] = jnp.zeros((bb, 1, 16, 128), F32)
    s_ref[:, 15:16, :, :] = jnp.zeros((bb, 1, 16, 128), F32)
    s_ref[:, :, 0:1, :] = jnp.zeros((bb, 16, 1, 128), F32)
    s_ref[:, :, 15:16, :] = jnp.zeros((bb, 16, 1, 128), F32)
    s_ref[:, 1:15, 1:15, :] = r2.reshape(bb, 14, 14, 128)
    acc = jnp.zeros((bb * 196, 4), F32)
    k = 0
    for ay in range(3):
        for ax in range(3):
            sl3 = s_ref[:, ay:ay + 14, ax:ax + 14, :].reshape(bb * 196, 128)
            acc = acc + jnp.dot(sl3.astype(BF16), m3s_ref[:, k, :],
                                preferred_element_type=F32)
            k += 1
    out = jax.nn.sigmoid(acc + b3_ref[0])
    out_ref[...] = out.reshape(bb, 14, 14, 4)


def kernel(inputs, conv1_W, conv1_b, conv2_W, conv2_b, enc_W, enc_b, embeddings,
           dec_W, dec_b, deconv1_W, deconv1_b, deconv2_W, deconv2_b,
           deconv3_W, deconv3_b):
    m1 = _build_m1(conv1_W).astype(BF16)
    k2 = _build_k2(conv2_W).astype(BF16)
    d1 = _build_d1(deconv1_W).astype(BF16)
    m2 = _build_m2(deconv2_W).astype(BF16)
    m3s = jnp.stack(_build_m3(deconv3_W), axis=1).astype(BF16)
    b1t = jnp.tile(conv1_b, 4).reshape(1, 128)
    b2t = conv2_b.reshape(1, 64)
    db1t = jnp.tile(deconv1_b, 4).reshape(1, 256)
    db2t = jnp.tile(deconv2_b, 4).reshape(1, 128)
    b3t = jnp.tile(deconv3_b, 4).reshape(1, 4)
    encWb = enc_W.astype(BF16)
    embT = embeddings.astype(BF16).T
    emb2 = jnp.sum(embeddings * embeddings, axis=1).reshape(1, 64)
    embQ = embeddings.astype(BF16)
    decWb = dec_W.astype(BF16)
    encbt = enc_b.reshape(1, 128)
    decbt = dec_b.reshape(1, 1568)

    x14 = _s2d(inputs)
    x14p = jnp.pad(x14, ((0, 0), (0, 2), (0, 2), (0, 0)))
    xq = _s2d(x14p)

    def _wfull(shape):
        return pl.BlockSpec(shape, lambda i: tuple(0 for _ in shape))

    BB1 = 256
    h = pl.pallas_call(
        _enc_conv_kernel,
        grid=(B // BB1,),
        in_specs=[
            pl.BlockSpec((BB1, 8, 8, 16), lambda i: (i, 0, 0, 0)),
            _wfull((64, 128)), _wfull((512, 64)),
            _wfull((1, 128)), _wfull((1, 64)),
        ],
        out_specs=pl.BlockSpec((BB1, 7, 7, 64), lambda i: (i, 0, 0, 0)),
        out_shape=jax.ShapeDtypeStruct((B, 7, 7, 64), F32),
        scratch_shapes=[pltpu.VMEM((BB1, 8, 8, 128), F32)],
    )(xq, m1, k2, b1t, b2t)

    flat = h.reshape(B, 3136)

    BB2 = 512
    d, loss2d = pl.pallas_call(
        _vq_kernel,
        grid=(B // BB2,),
        in_specs=[
            pl.BlockSpec((BB2, 3136), lambda i: (i, 0)),
            _wfull((3136, 128)), _wfull((1, 128)), _wfull((128, 64)),
            _wfull((1, 64)), _wfull((64, 128)), _wfull((128, 1568)),
            _wfull((1, 1568)),
        ],
        out_specs=[
            pl.BlockSpec((BB2, 1568), lambda i: (i, 0)),
            pl.BlockSpec((1, 1), lambda i: (0, 0), memory_space=pltpu.SMEM),
        ],
        out_shape=[
            jax.ShapeDtypeStruct((B, 1568), F32),
            jax.ShapeDtypeStruct((1, 1), F32),
        ],
    )(flat, encWb, encbt, embT, emb2, embQ, decWb, decbt)

    vq_loss = loss2d[0, 0]

    dsq = d.reshape(B, 7, 7, 32)
    dp = jnp.pad(dsq, ((0, 0), (1, 0), (1, 0), (0, 0)))
    BB3 = 256
    r1 = pl.pallas_call(
        _deconv1_kernel,
        grid=(B // BB3,),
        in_specs=[
            pl.BlockSpec((BB3, 8, 8, 32), lambda i: (i, 0, 0, 0)),
            _wfull((128, 256)), _wfull((1, 256)),
        ],
        out_specs=pl.BlockSpec((BB3, 7, 7, 256), lambda i: (i, 0, 0, 0)),
        out_shape=jax.ShapeDtypeStruct((B, 7, 7, 256), F32),
    )(dp, d1, db1t)

    r1img = r1.reshape(B, 7, 7, 2, 2, 64).transpose(0, 1, 3, 2, 4, 5).reshape(
        B, 14, 14, 64)
    r1p = jnp.pad(r1img, ((0, 0), (1, 0), (1, 0), (0, 0)))

    BB4 = 32
    out4 = pl.pallas_call(
        _dec_kernel,
        grid=(B // BB4,),
        in_specs=[
            pl.BlockSpec((BB4, 15, 15, 64), lambda i: (i, 0, 0, 0)),
            _wfull((256, 128)), _wfull((1, 128)),
            _wfull((128, 9, 4)), _wfull((1, 4)),
        ],
        out_specs=pl.BlockSpec((BB4, 14, 14, 4), lambda i: (i, 0, 0, 0)),
        out_shape=jax.ShapeDtypeStruct((B, 14, 14, 4), F32),
        scratch_shapes=[pltpu.VMEM((BB4, 16, 16, 128), F32)],
    )(r1p, m2, db2t, m3s, b3t)

    recon = out4.reshape(B, 14, 14, 2, 2).transpose(0, 1, 3, 2, 4).reshape(
        B, 28, 28, 1)
    return (recon, vq_loss)


# speedup vs baseline: 1.1381x; 1.1381x over previous
import jax
import jax.numpy as jnp
import numpy as np
from jax.experimental import pallas as pl
from jax.experimental.pallas import tpu as pltpu

F32 = jnp.float32
BF16 = jnp.bfloat16
B = 8192


def _s2d(x):
    b, h, w, c = x.shape
    return x.reshape(b, h // 2, 2, w // 2, 2, c).transpose(0, 1, 3, 2, 4, 5).reshape(
        b, h // 2, w // 2, 4 * c)


def _build_m1(W1):
    mats = []
    for Ay in (0, 1):
        for Ax in (0, 1):
            M = jnp.zeros((2, 2, 2, 2, 2, 2, 32), F32)
            for qy in (0, 1):
                for qx in (0, 1):
                    for p0y in (0, 1):
                        for p0x in (0, 1):
                            for py in (0, 1):
                                for px in (0, 1):
                                    ay = 2 * Ay + qy - py
                                    ax = 2 * Ax + qx - px
                                    if ay in (0, 1) and ax in (0, 1):
                                        ky, kx = 2 * ay + p0y, 2 * ax + p0x
                                        if ky <= 2 and kx <= 2:
                                            M = M.at[qy, qx, p0y, p0x, py, px, :].set(
                                                W1[ky, kx, 0, :])
            mats.append(M.reshape(16, 128))
    return mats


def _build_k2(W2):
    mats = []
    for Ay in (0, 1):
        for Ax in (0, 1):
            M = jnp.zeros((2, 2, 32, 64), F32)
            for py in (0, 1):
                for px in (0, 1):
                    ky, kx = 2 * Ay + py, 2 * Ax + px
                    if ky <= 2 and kx <= 2:
                        M = M.at[py, px].set(W2[ky, kx])
            mats.append(M.reshape(128, 64))
    return mats


def _build_d1(DW1):
    mats = []
    for Ay in (0, 1):
        for Ax in (0, 1):
            M = jnp.zeros((32, 2, 2, 64), F32)
            for ry in (0, 1):
                for rx in (0, 1):
                    ky, kx = 2 * Ay - ry, 2 * Ax - rx
                    if 0 <= ky <= 2 and 0 <= kx <= 2:
                        M = M.at[:, ry, rx, :].set(DW1[ky, kx])
            mats.append(M.reshape(32, 256))
    return mats


def _dc2_combos():
    out = []
    for w in (0, 1):
        for Ay in (0, 1):
            for sy in (0, 1):
                ky = 2 * Ay - sy
                if not (0 <= ky <= 2):
                    continue
                v = w + Ay - 1
                uy = -1 if v == -1 else 0
                ry = 1 if v != 0 else 0
                ty = 2 * w + sy
                out.append((uy, ry, ky, ty))
    return out


def _build_m2(DW2):
    combos = _dc2_combos()
    mats = {}
    for uy in (-1, 0):
        for ux in (-1, 0):
            M = jnp.zeros((2, 2, 64, 4, 4, 32), F32)
            for (cuy, ry, ky, ty) in combos:
                if cuy != uy:
                    continue
                for (cux, rx, kx, tx) in combos:
                    if cux != ux:
                        continue
                    M = M.at[ry, rx, :, ty, tx, :].add(DW2[ky, kx])
            mats[(uy, ux)] = M.reshape(256, 512)
    return mats


def _dc3_combos():
    out = []
    for tp in range(4):
        for dy in range(3):
            v = tp + dy - 1
            out.append((v // 4 if v >= 0 else -1, v % 4, dy, tp))
    return out


def _build_m3(W3):
    combos = _dc3_combos()
    mats = {}
    for Ey in (-1, 0, 1):
        for Ex in (-1, 0, 1):
            M = jnp.zeros((4, 4, 32, 4, 4), F32)
            for (cEy, ty, dy, tpy) in combos:
                if cEy != Ey:
                    continue
                for (cEx, tx, dx, tpx) in combos:
                    if cEx != Ex:
                        continue
                    M = M.at[ty, tx, :, tpy, tpx].add(W3[dy, dx, :, 0])
            mats[(Ey, Ex)] = M.reshape(512, 16)
    return mats


def _positions(m):
    i = jax.lax.broadcasted_iota(jnp.int32, (m, 1), 0)
    r = i % 49
    return r % 7, r // 7


def _tap(X, dy, dx, xpos, ypos):
    off = 7 * dy + dx
    Y = X if off == 0 else pltpu.roll(X, (-off) % X.shape[0], axis=0)
    ok = None
    if dx == 1:
        ok = xpos <= 5
    elif dx == -1:
        ok = xpos >= 1
    if dy == 1:
        oky = ypos <= 5
        ok = oky if ok is None else (ok & oky)
    elif dy == -1:
        oky = ypos >= 1
        ok = oky if ok is None else (ok & oky)
    if ok is None:
        return Y
    return jnp.where(ok, Y, jnp.zeros_like(Y))


def _enc_kernel(xq_ref, m1_ref, k2_ref, b1_ref, b2_ref, h_ref):
    m = xq_ref.shape[0]
    xpos, ypos = _positions(m)
    X = xq_ref[...]
    X1 = jnp.concatenate(
        [_tap(X, dy, dx, xpos, ypos) for dy in (0, 1) for dx in (0, 1)], axis=1)
    o1 = jnp.dot(X1, m1_ref[...], preferred_element_type=F32)
    o1 = jnp.maximum(o1 + b1_ref[0], 0.0).astype(BF16)
    X2 = jnp.concatenate(
        [_tap(o1, dy, dx, xpos, ypos) for dy in (0, 1) for dx in (0, 1)], axis=1)
    o2 = jnp.dot(X2, k2_ref[...], preferred_element_type=F32)
    h_ref[...] = jnp.maximum(o2 + b2_ref[0], 0.0).astype(BF16)


def _vq_kernel(flat_ref, encW_ref, encb_ref, embT_ref, emb2_ref, embQ_ref,
               decW_ref, decb_ref, d_ref, loss_ref):
    bb = flat_ref.shape[0]
    enc = jnp.dot(flat_ref[...], encW_ref[...],
                  preferred_element_type=F32) + encb_ref[0]
    cross = jnp.dot(enc.astype(BF16), embT_ref[...], preferred_element_type=F32)
    dist = jnp.sum(enc * enc, axis=1, keepdims=True) + emb2_ref[0] - 2.0 * cross
    idx = jnp.argmin(dist, axis=1)
    onehot = (jax.lax.broadcasted_iota(jnp.int32, (bb, 64), 1)
              == idx[:, None]).astype(BF16)
    q = jnp.dot(onehot, embQ_ref[...], preferred_element_type=F32)
    diff = q - enc
    part = jnp.sum(diff * diff) * (2.0 / (B * 128))

    @pl.when(pl.program_id(0) == 0)
    def _():
        loss_ref[0, 0] = 0.0

    loss_ref[0, 0] += part
    dd = jnp.dot(q.astype(BF16), decW_ref[...], preferred_element_type=F32)
    d_ref[...] = jnp.maximum(dd + decb_ref[0], 0.0).astype(BF16)


def _dec_kernel(d_ref, d1_ref, db1_ref, m2c_ref, m2e_ref, m2x_ref,
                m3c_ref, m3e_ref, db2_ref, b3_ref, out_ref):
    m = d_ref.shape[0]
    xpos, ypos = _positions(m)
    D = d_ref[...]
    X1 = jnp.concatenate(
        [_tap(D, dy, dx, xpos, ypos) for dy in (-1, 0) for dx in (-1, 0)], axis=1)
    r1 = jnp.dot(X1, d1_ref[...], preferred_element_type=F32)
    r1 = jnp.maximum(r1 + db1_ref[0], 0.0).astype(BF16)

    acc = jnp.dot(r1, m2c_ref[...], preferred_element_type=F32)
    e1 = _tap(r1[:, 192:256], -1, -1, xpos, ypos)
    e2 = _tap(r1[:, 128:256], -1, 0, xpos, ypos)
    upd = jnp.dot(jnp.concatenate([e1, e2], axis=1), m2e_ref[...],
                  preferred_element_type=F32)
    x1 = _tap(jnp.concatenate([r1[:, 64:128], r1[:, 192:256]], axis=1),
              0, -1, xpos, ypos)
    updx = jnp.dot(x1, m2x_ref[...], preferred_element_type=F32)
    updx_w = jnp.concatenate(
        [updx.reshape(m, 4, 1, 32), jnp.zeros((m, 4, 3, 32), F32)],
        axis=2).reshape(m, 512)
    acc = jnp.concatenate([acc[:, 0:128] + upd, acc[:, 128:512]],
                          axis=1) + updx_w
    r2 = jnp.maximum(acc + db2_ref[0], 0.0).astype(BF16)

    o = jnp.dot(r2, m3c_ref[...], preferred_element_type=F32)
    r2v = r2.reshape(m, 4, 4, 32)
    pieces = []
    for (Ey, Ex) in ((0, -1), (0, 1), (-1, 0), (1, 0), (-1, -1), (-1, 1),
                     (1, -1), (1, 1)):
        if Ey == 0:
            tx = 3 if Ex == -1 else 0
            src = r2v[:, :, tx, :].reshape(m, 128)
        elif Ex == 0:
            ty = 3 if Ey == -1 else 0
            src = r2v[:, ty, :, :].reshape(m, 128)
        else:
            ty = 3 if Ey == -1 else 0
            tx = 3 if Ex == -1 else 0
            src = r2v[:, ty, tx, :].reshape(m, 32)
        pieces.append(_tap(src, Ey, Ex, xpos, ypos))
    o = o + jnp.dot(jnp.concatenate(pieces, axis=1), m3e_ref[...],
                    preferred_element_type=F32)
    out_ref[...] = jax.nn.sigmoid(o + b3_ref[0])


def kernel(inputs, conv1_W, conv1_b, conv2_W, conv2_b, enc_W, enc_b, embeddings,
           dec_W, dec_b, deconv1_W, deconv1_b, deconv2_W, deconv2_b,
           deconv3_W, deconv3_b):
    m1 = jnp.concatenate(_build_m1(conv1_W), axis=0).astype(BF16)
    k2 = jnp.concatenate(_build_k2(conv2_W), axis=0).astype(BF16)
    d1l = _build_d1(deconv1_W)
    d1 = jnp.concatenate(d1l, axis=0).astype(BF16)
    m2d = _build_m2(deconv2_W)
    m2c = m2d[(0, 0)].astype(BF16)
    m2e = jnp.concatenate([m2d[(-1, -1)][192:256, 0:128],
                           m2d[(-1, 0)][128:256, 0:128]], axis=0).astype(BF16)
    m2x_full = m2d[(0, -1)].reshape(2, 2, 64, 4, 4, 32)
    m2x = jnp.concatenate([m2x_full[0, 1], m2x_full[1, 1]],
                          axis=0)[:, :, 0, :].reshape(128, 128).astype(BF16)
    m3d = _build_m3(deconv3_W)
    m3c = m3d[(0, 0)].astype(BF16)
    ep = []
    for (Ey, Ex) in ((0, -1), (0, 1), (-1, 0), (1, 0), (-1, -1), (-1, 1),
                     (1, -1), (1, 1)):
        Mf = m3d[(Ey, Ex)].reshape(4, 4, 32, 16)
        if Ey == 0:
            tx = 3 if Ex == -1 else 0
            ep.append(Mf[:, tx, :, :].reshape(128, 16))
        elif Ex == 0:
            ty = 3 if Ey == -1 else 0
            ep.append(Mf[ty, :, :, :].reshape(128, 16))
        else:
            ty = 3 if Ey == -1 else 0
            tx = 3 if Ex == -1 else 0
            ep.append(Mf[ty, tx, :, :].reshape(32, 16))
    m3e = jnp.concatenate(ep, axis=0).astype(BF16)

    b1t = jnp.tile(conv1_b, 4).reshape(1, 128)
    b2t = conv2_b.reshape(1, 64)
    db1t = jnp.tile(deconv1_b, 4).reshape(1, 256)
    db2t = jnp.tile(deconv2_b, 16).reshape(1, 512)
    b3t = jnp.tile(deconv3_b, 16).reshape(1, 16)
    encWb = enc_W.astype(BF16)
    embT = embeddings.astype(BF16).T
    emb2 = jnp.sum(embeddings * embeddings, axis=1).reshape(1, 64)
    embQ = embeddings.astype(BF16)
    decWb = dec_W.astype(BF16)
    encbt = enc_b.reshape(1, 128)
    decbt = dec_b.reshape(1, 1568)

    x14 = _s2d(inputs)
    x14p = jnp.pad(x14, ((0, 0), (0, 2), (0, 2), (0, 0)))
    xq = _s2d(x14p)[:, :7, :7, :]
    xq49 = xq.reshape(B * 49, 16).astype(BF16)

    def _wfull(shape):
        return pl.BlockSpec(shape, lambda i: tuple(0 for _ in shape))

    BB1 = 256
    M1r = BB1 * 49
    h = pl.pallas_call(
        _enc_kernel,
        grid=(B // BB1,),
        in_specs=[
            pl.BlockSpec((M1r, 16), lambda i: (i, 0)),
            _wfull((64, 128)), _wfull((512, 64)),
            _wfull((1, 128)), _wfull((1, 64)),
        ],
        out_specs=pl.BlockSpec((M1r, 64), lambda i: (i, 0)),
        out_shape=jax.ShapeDtypeStruct((B * 49, 64), BF16),
    )(xq49, m1, k2, b1t, b2t)

    flat = h.reshape(B, 3136)

    BB2 = 2048
    d, loss2d = pl.pallas_call(
        _vq_kernel,
        grid=(B // BB2,),
        in_specs=[
            pl.BlockSpec((BB2, 3136), lambda i: (i, 0)),
            _wfull((3136, 128)), _wfull((1, 128)), _wfull((128, 64)),
            _wfull((1, 64)), _wfull((64, 128)), _wfull((128, 1568)),
            _wfull((1, 1568)),
        ],
        out_specs=[
            pl.BlockSpec((BB2, 1568), lambda i: (i, 0)),
            pl.BlockSpec((1, 1), lambda i: (0, 0), memory_space=pltpu.SMEM),
        ],
        out_shape=[
            jax.ShapeDtypeStruct((B, 1568), BF16),
            jax.ShapeDtypeStruct((1, 1), F32),
        ],
    )(flat, encWb, encbt, embT, emb2, embQ, decWb, decbt)

    vq_loss = loss2d[0, 0]
    d49 = d.reshape(B * 49, 32)

    BB3 = 32
    M3r = BB3 * 49
    out = pl.pallas_call(
        _dec_kernel,
        grid=(B // BB3,),
        in_specs=[
            pl.BlockSpec((M3r, 32), lambda i: (i, 0)),
            _wfull((128, 256)), _wfull((1, 256)),
            _wfull((256, 512)), _wfull((192, 128)), _wfull((128, 128)),
            _wfull((512, 16)), _wfull((640, 16)),
            _wfull((1, 512)), _wfull((1, 16)),
        ],
        out_specs=pl.BlockSpec((M3r, 16), lambda i: (i, 0)),
        out_shape=jax.ShapeDtypeStruct((B * 49, 16), F32),
    )(d49, d1, db1t, m2c, m2e, m2x, m3c, m3e, db2t, b3t)

    recon = out.reshape(B, 7, 7, 4, 4).transpose(0, 1, 3, 2, 4).reshape(
        B, 28, 28, 1)
    return (recon, vq_loss)
